# Initial kernel scaffold; baseline (speedup 1.0000x reference)
import functools

import jax
import jax.numpy as jnp
from jax import lax
from jax.experimental import pallas as pl
from jax.experimental.pallas import tpu as pltpu
from jax.experimental.pallas import tpu_sc as plsc

_NC = 2
_NS = 16
_NW = _NC * _NS
_CH = 128


@functools.lru_cache(maxsize=None)
def _make_sc_agg(n_rows, sr, nch, d, with_deg):
    zr = sr // _NS
    zc = zr // _CH
    mesh = plsc.VectorSubcoreMesh(core_axis_name="c", subcore_axis_name="s")

    out_type = [jax.ShapeDtypeStruct((_NC, sr, d), jnp.float32)]
    scratch = [
        pltpu.VMEM((nch, _CH), jnp.int32),
        pltpu.VMEM((nch, _CH), jnp.int32),
        pltpu.VMEM((_CH, d), jnp.float32),
        pltpu.VMEM_SHARED((sr, d), jnp.float32),
        pltpu.SemaphoreType.DMA,
    ]
    if with_deg:
        out_type.append(jax.ShapeDtypeStruct((_NC, sr, 16), jnp.float32))
        scratch += [
            pltpu.VMEM((_CH, 16), jnp.float32),
            pltpu.VMEM((zr, 16), jnp.float32),
            pltpu.VMEM_SHARED((sr, 16), jnp.float32),
        ]

    def body(*refs):
        if with_deg:
            (h_hbm, src_hbm, dst_hbm, z_hbm, z16_hbm, ones_hbm,
             agg_out, deg_out,
             src_v, dst_v, rows_v, agg_sh, sem, ones_v, d16_v, deg_sh) = refs
        else:
            (h_hbm, src_hbm, dst_hbm, z_hbm,
             agg_out,
             src_v, dst_v, rows_v, agg_sh, sem) = refs
        cid = lax.axis_index("c")
        sid = lax.axis_index("s")
        wid = cid * _NS + sid

        pltpu.sync_copy(src_hbm.at[wid], src_v)
        pltpu.sync_copy(dst_hbm.at[wid], dst_v)
        pltpu.sync_copy(z_hbm, rows_v)
        for k in range(zc):
            pltpu.sync_copy(rows_v, agg_sh.at[pl.ds((sid * zc + k) * _CH, _CH), :])
        if with_deg:
            pltpu.sync_copy(z16_hbm, d16_v)
            pltpu.sync_copy(d16_v, deg_sh.at[pl.ds(sid * zr, zr), :])
            pltpu.sync_copy(ones_hbm, ones_v)
        plsc.subcore_barrier()

        def step(j, carry):
            pltpu.async_copy(h_hbm.at[src_v.at[j]], rows_v, sem).wait()
            pltpu.sync_copy(rows_v, agg_sh.at[dst_v.at[j]], add=True)
            if with_deg:
                pltpu.sync_copy(ones_v, deg_sh.at[dst_v.at[j]], add=True)
            return carry

        lax.fori_loop(0, nch, step, 0)
        plsc.subcore_barrier()

        for k in range(zc):
            r0 = (sid * zc + k) * _CH
            pltpu.sync_copy(agg_sh.at[pl.ds(r0, _CH), :], rows_v)
            pltpu.sync_copy(rows_v, agg_out.at[cid, pl.ds(r0, _CH), :])
        if with_deg:
            pltpu.sync_copy(deg_sh.at[pl.ds(sid * zr, zr), :], d16_v)
            pltpu.sync_copy(d16_v, deg_out.at[cid, pl.ds(sid * zr, zr), :])

    return pl.kernel(body, out_type=out_type, mesh=mesh, scratch_types=scratch)


_TC_PARAMS = pltpu.CompilerParams(vmem_limit_bytes=100 * 1024 * 1024)


def _emb(x, w, b):
    def body(x_ref, w_ref, b_ref, o_ref):
        o_ref[...] = (
            jnp.dot(x_ref[...], w_ref[...], preferred_element_type=jnp.float32)
            + b_ref[...]
        )

    return pl.pallas_call(
        body,
        out_shape=jax.ShapeDtypeStruct((x.shape[0], w.shape[1]), jnp.float32),
        compiler_params=_TC_PARAMS,
    )(x, w, b)


def _layer_math(h, ap_ref, dp_ref, ws, wn, bias, gam, bet, n):
    psum = ap_ref[0, :n, :] + ap_ref[1, :n, :]
    deg = jnp.maximum(dp_ref[0, :n, 0:1] + dp_ref[1, :n, 0:1], 1.0)
    agg = psum / deg
    hh = (
        jnp.dot(h, ws, preferred_element_type=jnp.float32)
        + jnp.dot(agg, wn, preferred_element_type=jnp.float32)
        + bias
    )
    mean = jnp.mean(hh, axis=0, keepdims=True)
    c = hh - mean
    var = jnp.mean(c * c, axis=0, keepdims=True)
    hh = gam * c * lax.rsqrt(var + 1e-5) + bet
    return h + jnp.maximum(hh, 0.0)


def _layer(h, agg_p, deg_p, ws, wn, bias, gam, bet):
    n = h.shape[0]

    def body(h_ref, ap_ref, dp_ref, ws_ref, wn_ref, b_ref, g_ref, be_ref, o_ref):
        o_ref[...] = _layer_math(
            h_ref[...], ap_ref, dp_ref, ws_ref[...], wn_ref[...],
            b_ref[...], g_ref[...], be_ref[...], n,
        )

    return pl.pallas_call(
        body,
        out_shape=jax.ShapeDtypeStruct(h.shape, jnp.float32),
        compiler_params=_TC_PARAMS,
    )(h, agg_p, deg_p, ws, wn, bias, gam, bet)


def _final(h, agg_p, deg_p, ws, wn, bias, gam, bet, batch2d, ng, mlp):
    n = h.shape[0]
    nc_out = mlp[2][0].shape[1]

    def body(h_ref, ap_ref, dp_ref, ws_ref, wn_ref, b_ref, g_ref, be_ref,
             batch_ref, w1_ref, b1_ref, w2_ref, b2_ref, w3_ref, b3_ref, o_ref):
        h4 = _layer_math(
            h_ref[...], ap_ref, dp_ref, ws_ref[...], wn_ref[...],
            b_ref[...], g_ref[...], be_ref[...], n,
        )
        groups = lax.broadcasted_iota(jnp.int32, (ng, n), 0)
        m = (batch_ref[...] == groups).astype(jnp.float32)
        g = jnp.dot(m, h4, preferred_element_type=jnp.float32)
        g = jnp.maximum(
            jnp.dot(g, w1_ref[...], preferred_element_type=jnp.float32)
            + b1_ref[...], 0.0)
        g = jnp.maximum(
            jnp.dot(g, w2_ref[...], preferred_element_type=jnp.float32)
            + b2_ref[...], 0.0)
        o_ref[...] = (
            jnp.dot(g, w3_ref[...], preferred_element_type=jnp.float32)
            + b3_ref[...]
        )

    return pl.pallas_call(
        body,
        out_shape=jax.ShapeDtypeStruct((ng, nc_out), jnp.float32),
        compiler_params=_TC_PARAMS,
    )(h, agg_p, deg_p, ws, wn, bias, gam, bet, batch2d,
      mlp[0][0], mlp[0][1].reshape(1, -1),
      mlp[1][0], mlp[1][1].reshape(1, -1),
      mlp[2][0], mlp[2][1].reshape(1, -1))


def kernel(x, edge_index, batch, params):
    n, _ = x.shape
    e = edge_index.shape[1]
    d = params["emb_W"].shape[1]
    ng = 64

    ept = -(-e // (_NW * _CH)) * _CH
    nch = ept // _CH
    epad = _NW * ept - e
    src = jnp.concatenate([edge_index[0], jnp.zeros((epad,), jnp.int32)])
    dst = jnp.concatenate([edge_index[1], jnp.full((epad,), n, jnp.int32)])
    src3 = src.reshape(_NW, nch, _CH)
    dst3 = dst.reshape(_NW, nch, _CH)

    sr = -(-(n + 1) // (_NS * _CH)) * (_NS * _CH)
    zr = sr // _NS
    z128 = jnp.zeros((_CH, d), jnp.float32)
    z16 = jnp.zeros((zr, 16), jnp.float32)
    ones16 = jnp.ones((_CH, 16), jnp.float32)

    h = _emb(x, params["emb_W"], params["emb_b"].reshape(1, -1))

    sc_deg = _make_sc_agg(n, sr, nch, d, True)
    sc = _make_sc_agg(n, sr, nch, d, False)

    agg_p, deg_p = sc_deg(h, src3, dst3, z128, z16, ones16)
    batch2d = batch.reshape(1, n)

    out = None
    for li, lp in enumerate(params["layers"]):
        bias = (lp["b_self"] + lp["b_neigh"]).reshape(1, -1)
        gam = lp["gamma"].reshape(1, -1)
        bet = lp["beta"].reshape(1, -1)
        if li < len(params["layers"]) - 1:
            h = _layer(h, agg_p, deg_p, lp["W_self"], lp["W_neigh"], bias, gam, bet)
            (agg_p,) = sc(h, src3, dst3, z128)
        else:
            out = _final(h, agg_p, deg_p, lp["W_self"], lp["W_neigh"], bias,
                         gam, bet, batch2d, ng, params["mlp"])
    return out


# speedup vs baseline: 4.7852x; 4.7852x over previous
import functools

import jax
import jax.numpy as jnp
from jax import lax
from jax.experimental import pallas as pl
from jax.experimental.pallas import tpu as pltpu
from jax.experimental.pallas import tpu_sc as plsc

_NC = 2
_NS = 16
_NW = _NC * _NS
_CH = 128


@functools.lru_cache(maxsize=None)
def _make_sc_agg(sr, nch, d):
    zc = sr // _NS // _CH
    mesh = plsc.VectorSubcoreMesh(core_axis_name="c", subcore_axis_name="s", num_cores=_NC, num_subcores=_NS)

    out_type = jax.ShapeDtypeStruct((_NC, sr, d), jnp.float32)
    scratch = [
        pltpu.VMEM((nch, _CH), jnp.int32),
        pltpu.VMEM((nch, _CH), jnp.int32),
        pltpu.VMEM((_CH, d), jnp.float32),
        pltpu.VMEM_SHARED((sr, d), jnp.float32),
        pltpu.SemaphoreType.DMA,
    ]

    def body(h_hbm, src_hbm, dst_hbm, z_hbm, agg_out,
             src_v, dst_v, rows_v, agg_sh, sem):
        cid = lax.axis_index("c")
        sid = lax.axis_index("s")
        wid = cid * _NS + sid

        pltpu.sync_copy(src_hbm.at[wid], src_v)
        pltpu.sync_copy(dst_hbm.at[wid], dst_v)
        pltpu.sync_copy(z_hbm, rows_v)
        for k in range(zc):
            pltpu.sync_copy(rows_v, agg_sh.at[pl.ds((sid * zc + k) * _CH, _CH), :])
        plsc.subcore_barrier()

        def step(j, carry):
            pltpu.async_copy(h_hbm.at[src_v.at[j]], rows_v, sem).wait()
            pltpu.sync_copy(rows_v, agg_sh.at[dst_v.at[j]], add=True)
            return carry

        lax.fori_loop(0, nch, step, 0)
        plsc.subcore_barrier()

        for k in range(zc):
            r0 = (sid * zc + k) * _CH
            pltpu.sync_copy(agg_sh.at[pl.ds(r0, _CH), :], rows_v)
            pltpu.sync_copy(rows_v, agg_out.at[cid, pl.ds(r0, _CH), :])

    return pl.kernel(body, out_type=out_type, mesh=mesh, scratch_types=scratch)


@functools.lru_cache(maxsize=None)
def _make_sc_deg(sr, nch, d):
    zc = sr // _NS // _CH
    mesh = plsc.VectorSubcoreMesh(core_axis_name="c", subcore_axis_name="s", num_cores=_NC, num_subcores=_NS)

    out_type = jax.ShapeDtypeStruct((_NC, sr, d), jnp.float32)
    scratch = [
        pltpu.VMEM((nch, _CH), jnp.int32),
        pltpu.VMEM((_CH, d), jnp.float32),
        pltpu.VMEM((_CH, d), jnp.float32),
        pltpu.VMEM_SHARED((sr, d), jnp.float32),
    ]

    def body(dst_hbm, z_hbm, ones_hbm, deg_out, dst_v, ones_v, zb_v, deg_sh):
        cid = lax.axis_index("c")
        sid = lax.axis_index("s")
        wid = cid * _NS + sid

        pltpu.sync_copy(dst_hbm.at[wid], dst_v)
        pltpu.sync_copy(z_hbm, zb_v)
        for k in range(zc):
            pltpu.sync_copy(zb_v, deg_sh.at[pl.ds((sid * zc + k) * _CH, _CH), :])
        pltpu.sync_copy(ones_hbm, ones_v)
        plsc.subcore_barrier()

        def step(j, carry):
            pltpu.sync_copy(ones_v, deg_sh.at[dst_v.at[j]], add=True)
            return carry

        lax.fori_loop(0, nch, step, 0)
        plsc.subcore_barrier()

        for k in range(zc):
            r0 = (sid * zc + k) * _CH
            pltpu.sync_copy(deg_sh.at[pl.ds(r0, _CH), :], zb_v)
            pltpu.sync_copy(zb_v, deg_out.at[cid, pl.ds(r0, _CH), :])

    return pl.kernel(body, out_type=out_type, mesh=mesh, scratch_types=scratch)


_TC_PARAMS = pltpu.CompilerParams(vmem_limit_bytes=100 * 1024 * 1024)


def _emb(x, w, b):
    def body(x_ref, w_ref, b_ref, o_ref):
        o_ref[...] = (
            jnp.dot(x_ref[...], w_ref[...], preferred_element_type=jnp.float32)
            + b_ref[...]
        )

    return pl.pallas_call(
        body,
        out_shape=jax.ShapeDtypeStruct((x.shape[0], w.shape[1]), jnp.float32),
        compiler_params=_TC_PARAMS,
    )(x, w, b)


def _layer_math(h, ap_ref, dp_ref, ws, wn, bias, gam, bet, n):
    psum = ap_ref[0, :n, :] + ap_ref[1, :n, :]
    deg = jnp.maximum(dp_ref[0, :n, 0:1] + dp_ref[1, :n, 0:1], 1.0)
    agg = psum / deg
    hh = (
        jnp.dot(h, ws, preferred_element_type=jnp.float32)
        + jnp.dot(agg, wn, preferred_element_type=jnp.float32)
        + bias
    )
    mean = jnp.mean(hh, axis=0, keepdims=True)
    c = hh - mean
    var = jnp.mean(c * c, axis=0, keepdims=True)
    hh = gam * c * lax.rsqrt(var + 1e-5) + bet
    return h + jnp.maximum(hh, 0.0)


def _layer(h, agg_p, deg_p, ws, wn, bias, gam, bet):
    n = h.shape[0]

    def body(h_ref, ap_ref, dp_ref, ws_ref, wn_ref, b_ref, g_ref, be_ref, o_ref):
        o_ref[...] = _layer_math(
            h_ref[...], ap_ref, dp_ref, ws_ref[...], wn_ref[...],
            b_ref[...], g_ref[...], be_ref[...], n,
        )

    return pl.pallas_call(
        body,
        out_shape=jax.ShapeDtypeStruct(h.shape, jnp.float32),
        compiler_params=_TC_PARAMS,
    )(h, agg_p, deg_p, ws, wn, bias, gam, bet)


def _final(h, agg_p, deg_p, ws, wn, bias, gam, bet, batch2d, ng, mlp):
    n = h.shape[0]
    nc_out = mlp[2][0].shape[1]

    def body(h_ref, ap_ref, dp_ref, ws_ref, wn_ref, b_ref, g_ref, be_ref,
             batch_ref, w1_ref, b1_ref, w2_ref, b2_ref, w3_ref, b3_ref, o_ref):
        h4 = _layer_math(
            h_ref[...], ap_ref, dp_ref, ws_ref[...], wn_ref[...],
            b_ref[...], g_ref[...], be_ref[...], n,
        )
        groups = lax.broadcasted_iota(jnp.int32, (ng, n), 0)
        m = (batch_ref[...] == groups).astype(jnp.float32)
        g = jnp.dot(m, h4, preferred_element_type=jnp.float32)
        g = jnp.maximum(
            jnp.dot(g, w1_ref[...], preferred_element_type=jnp.float32)
            + b1_ref[...], 0.0)
        g = jnp.maximum(
            jnp.dot(g, w2_ref[...], preferred_element_type=jnp.float32)
            + b2_ref[...], 0.0)
        o_ref[...] = (
            jnp.dot(g, w3_ref[...], preferred_element_type=jnp.float32)
            + b3_ref[...]
        )

    return pl.pallas_call(
        body,
        out_shape=jax.ShapeDtypeStruct((ng, nc_out), jnp.float32),
        compiler_params=_TC_PARAMS,
    )(h, agg_p, deg_p, ws, wn, bias, gam, bet, batch2d,
      mlp[0][0], mlp[0][1].reshape(1, -1),
      mlp[1][0], mlp[1][1].reshape(1, -1),
      mlp[2][0], mlp[2][1].reshape(1, -1))


def kernel(x, edge_index, batch, params):
    n, _ = x.shape
    e = edge_index.shape[1]
    d = params["emb_W"].shape[1]
    ng = 64

    ept = -(-e // (_NW * _CH)) * _CH
    nch = ept // _CH
    epad = _NW * ept - e
    src = jnp.concatenate([edge_index[0], jnp.zeros((epad,), jnp.int32)])
    dst = jnp.concatenate([edge_index[1], jnp.full((epad,), n, jnp.int32)])
    src3 = src.reshape(_NW, nch, _CH)
    dst3 = dst.reshape(_NW, nch, _CH)

    sr = -(-(n + 1) // (_NS * _CH)) * (_NS * _CH)
    z128 = jnp.zeros((_CH, d), jnp.float32)
    ones128 = jnp.ones((_CH, d), jnp.float32)

    h = _emb(x, params["emb_W"], params["emb_b"].reshape(1, -1))

    sc = _make_sc_agg(sr, nch, d)
    deg_p = _make_sc_deg(sr, nch, d)(dst3, z128, ones128)
    batch2d = batch.reshape(1, n)

    out = None
    for li, lp in enumerate(params["layers"]):
        bias = (lp["b_self"] + lp["b_neigh"]).reshape(1, -1)
        gam = lp["gamma"].reshape(1, -1)
        bet = lp["beta"].reshape(1, -1)
        agg_p = sc(h, src3, dst3, z128)
        if li < len(params["layers"]) - 1:
            h = _layer(h, agg_p, deg_p, lp["W_self"], lp["W_neigh"], bias, gam, bet)
        else:
            out = _final(h, agg_p, deg_p, lp["W_self"], lp["W_neigh"], bias,
                         gam, bet, batch2d, ng, params["mlp"])
    return out
